# Initial kernel scaffold; baseline (speedup 1.0000x reference)
import jax
import jax.numpy as jnp
from jax.experimental import pallas as pl


def kernel(x, fc_weight, bias):
    raise NotImplementedError("write your pallas kernel here")


# speedup vs baseline: 1.4339x; 1.4339x over previous
import functools

import jax
import jax.numpy as jnp
from jax import lax
from jax.experimental import pallas as pl
from jax.experimental.pallas import tpu as pltpu
from jax.experimental.pallas import tpu_sc as plsc

L = 16
NC, NS = 2, 16
NW = NC * NS
BATCH = 16384
FIELDS = 26
BPW = BATCH // NW
KPW = BPW * FIELDS
IDX_MINOR = 128
ROWS = KPW // IDX_MINOR


def _sc_call(xt, table, bias, *, interpret=False):
    mesh = plsc.VectorSubcoreMesh(
        core_axis_name="c", subcore_axis_name="s", num_cores=NC, num_subcores=NS
    )

    @functools.partial(
        pl.kernel,
        out_type=jax.ShapeDtypeStruct((BATCH,), jnp.float32),
        mesh=mesh,
        scratch_types=[
            pltpu.VMEM((KPW,), jnp.int32),
            pltpu.VMEM((KPW,), jnp.float32),
            pltpu.VMEM((BPW,), jnp.float32),
            pltpu.VMEM((L,), jnp.float32),
            pltpu.SemaphoreType.DMA,
        ],
        interpret=interpret,
    )
    def k(x_hbm, table_hbm, bias_hbm, out_hbm, idx_v, val_v, out_v, bias_v, sem):
        wid = lax.axis_index("s") * NC + lax.axis_index("c")
        pltpu.sync_copy(x_hbm.at[wid], idx_v)
        pltpu.sync_copy(bias_hbm, bias_v)
        pltpu.async_copy(table_hbm.at[idx_v], val_v, sem).wait()
        bvec = bias_v[...]
        for c in range(BPW // L):
            acc = bvec
            for f in range(FIELDS):
                acc = acc + val_v[pl.ds(f * BPW + c * L, L)]
            out_v[pl.ds(c * L, L)] = acc
        pltpu.sync_copy(out_v, out_hbm.at[pl.ds(wid * BPW, BPW)])

    return k(xt, table, bias)


def kernel(x, fc_weight, bias):
    xt = (
        x.reshape(NW, BPW, FIELDS)
        .transpose(0, 2, 1)
        .reshape(NW, KPW)
    )
    table = fc_weight.reshape(-1)
    bias16 = jnp.broadcast_to(bias.reshape(()), (L,))
    out = _sc_call(xt, table, bias16)
    return out.reshape(BATCH, 1)
